# baseline (speedup 1.0000x reference)
import functools
import math

import jax
import jax.numpy as jnp
from jax import lax
from jax.experimental import pallas as pl
from jax.experimental.pallas import tpu as pltpu
from jax.experimental.pallas import tpu_sc as plsc

EMB = 64
LANES = 16
NC = 2
NS = 16
NW = NC * NS
BLK = 128
NBUF = 5


def _pe_table(seq: int) -> jax.Array:
    den = jnp.exp(-jnp.arange(0, EMB, 2, dtype=jnp.float32)
                  * (math.log(10000.0) / EMB))
    pos = jnp.arange(0, seq, dtype=jnp.float32).reshape(seq, 1)
    pe = jnp.zeros((seq, EMB), dtype=jnp.float32)
    pe = pe.at[:, 0::2].set(jnp.sin(pos * den))
    pe = pe.at[:, 1::2].set(jnp.cos(pos * den))
    return pe


def _sc_body(nblk, nblk_w, batch, table_hbm, idx_hbm, pe_hbm, out_hbm,
             idx_v, pe_v, rin, rout, gsem, osem):
    wid = lax.axis_index("s") * NC + lax.axis_index("c")
    blk0 = wid * nblk_w
    row0 = blk0 * BLK

    pltpu.sync_copy(idx_hbm.at[wid], idx_v)
    pltpu.sync_copy(pe_hbm, pe_v)

    def gather(k, j):
        return pltpu.make_async_copy(
            table_hbm.at[idx_v.at[k]], rin[j], gsem[j])

    def put(k, j):
        return pltpu.make_async_copy(
            rout[j], out_hbm.at[pl.ds(row0 // 2 + k * (BLK // 2), BLK // 2)],
            osem[j])

    for j in range(NBUF):
        gather(j, j).start()

    def outer(kk, carry):
        for j in range(NBUF):
            k = kk * NBUF + j
            gather(k, j).wait()

            @pl.when(kk >= 1)
            def _():
                put(k, j).wait()

            s = (row0 + k * BLK) // batch
            pbase = 64 * (s % 2)
            pvec = [pe_v[s // 2, pl.ds(pbase + LANES * t, LANES)]
                    for t in range(EMB // LANES)]

            def row(rr, rc):
                for u in range(2 * EMB // LANES):
                    t = u % (EMB // LANES)
                    r = 2 * rr + u // (EMB // LANES)
                    rout[j][rr, pl.ds(LANES * u, LANES)] = (
                        rin[j][r, pl.ds(LANES * t, LANES)] * 8.0 + pvec[t])
                return rc

            lax.fori_loop(0, BLK // 2, row, 0, unroll=4)

            put(k, j).start()

            @pl.when(kk < nblk_w // NBUF - 1)
            def _():
                gather(k + NBUF, j).start()
        return carry

    lax.fori_loop(0, nblk_w // NBUF, outer, 0)

    for j in range(NBUF):
        put(0, j).wait()


def kernel(x, table):
    seq, batch = x.shape
    vocab, emb = table.shape
    assert emb == EMB
    n = seq * batch
    assert n % (NW * BLK * NBUF) == 0 and batch % BLK == 0
    nblk = n // BLK
    nblk_w = nblk // NW

    pe = _pe_table(seq).reshape(seq // 2, 2 * EMB)
    idx = x.reshape(NW, nblk_w, BLK)

    table = lax.optimization_barrier(table.reshape(-1)).reshape(vocab, emb)

    mesh = plsc.VectorSubcoreMesh(core_axis_name="c", subcore_axis_name="s")
    run = pl.kernel(
        functools.partial(_sc_body, nblk, nblk_w, batch),
        out_type=jax.ShapeDtypeStruct((n * EMB // 128, 128), jnp.float32),
        mesh=mesh,
        compiler_params=pltpu.CompilerParams(use_tc_tiling_on_sc=False),
        scratch_types=[
            pltpu.VMEM((nblk_w, BLK), jnp.int32),
            pltpu.VMEM((seq // 2, 2 * EMB), jnp.float32),
            [pltpu.VMEM((BLK, EMB), jnp.float32)] * NBUF,
            [pltpu.VMEM((BLK // 2, 128), jnp.float32)] * NBUF,
            [pltpu.SemaphoreType.DMA] * NBUF,
            [pltpu.SemaphoreType.DMA] * NBUF,
        ],
    )
    out = run(table, idx, pe)
    return out.reshape(seq, batch, EMB)


# speedup vs baseline: 1.1528x; 1.1528x over previous
import functools
import math

import jax
import jax.numpy as jnp
from jax import lax
from jax.experimental import pallas as pl
from jax.experimental.pallas import tpu as pltpu
from jax.experimental.pallas import tpu_sc as plsc

EMB = 64
LANES = 16
NC = 2
NS = 16
NW = NC * NS
BLK = 128
NBUF = 5


def _pe_table(seq: int) -> jax.Array:
    den = jnp.exp(-jnp.arange(0, EMB, 2, dtype=jnp.float32)
                  * (math.log(10000.0) / EMB))
    pos = jnp.arange(0, seq, dtype=jnp.float32).reshape(seq, 1)
    pe = jnp.zeros((seq, EMB), dtype=jnp.float32)
    pe = pe.at[:, 0::2].set(jnp.sin(pos * den))
    pe = pe.at[:, 1::2].set(jnp.cos(pos * den))
    return pe


def _sc_body(nblk, nblk_w, batch, table_hbm, idx_hbm, pe_hbm, out_hbm,
             idx_v, pe_v, rin, rout, gsem, osem):
    wid = lax.axis_index("s") * NC + lax.axis_index("c")
    blk0 = wid * nblk_w
    row0 = blk0 * BLK

    pltpu.sync_copy(idx_hbm.at[wid], idx_v)
    pltpu.sync_copy(pe_hbm, pe_v)

    def gather(k, j):
        return pltpu.make_async_copy(
            table_hbm.at[idx_v.at[k]], rin[j], gsem[j])

    def put(k, j):
        return pltpu.make_async_copy(
            rout[j], out_hbm.at[pl.ds(row0 // 2 + k * (BLK // 2), BLK // 2)],
            osem[j])

    for j in range(NBUF):
        gather(j, j).start()

    def outer(kk, carry):
        for j in range(NBUF):
            k = kk * NBUF + j
            gather(k, j).wait()

            @pl.when(kk >= 1)
            def _():
                put(k, j).wait()

            s = (row0 + k * BLK) // batch
            pbase = 64 * (s % 2)
            pvec = [pe_v[s // 2, pl.ds(pbase + LANES * t, LANES)]
                    for t in range(EMB // LANES)]

            @plsc.parallel_loop(0, BLK // 2, unroll=4)
            def _(rr):
                for u in range(2 * EMB // LANES):
                    t = u % (EMB // LANES)
                    r = 2 * rr + u // (EMB // LANES)
                    rout[j][rr, pl.ds(LANES * u, LANES)] = (
                        rin[j][r, pl.ds(LANES * t, LANES)] * 8.0 + pvec[t])

            put(k, j).start()

            @pl.when(kk < nblk_w // NBUF - 1)
            def _():
                gather(k + NBUF, j).start()
        return carry

    lax.fori_loop(0, nblk_w // NBUF, outer, 0)

    for j in range(NBUF):
        put(0, j).wait()


def kernel(x, table):
    seq, batch = x.shape
    vocab, emb = table.shape
    assert emb == EMB
    n = seq * batch
    assert n % (NW * BLK * NBUF) == 0 and batch % BLK == 0
    nblk = n // BLK
    nblk_w = nblk // NW

    pe = _pe_table(seq).reshape(seq // 2, 2 * EMB)
    idx = x.reshape(NW, nblk_w, BLK)

    table = lax.optimization_barrier(table.reshape(-1)).reshape(vocab, emb)

    mesh = plsc.VectorSubcoreMesh(core_axis_name="c", subcore_axis_name="s")
    run = pl.kernel(
        functools.partial(_sc_body, nblk, nblk_w, batch),
        out_type=jax.ShapeDtypeStruct((n * EMB // 128, 128), jnp.float32),
        mesh=mesh,
        compiler_params=pltpu.CompilerParams(use_tc_tiling_on_sc=False),
        scratch_types=[
            pltpu.VMEM((nblk_w, BLK), jnp.int32),
            pltpu.VMEM((seq // 2, 2 * EMB), jnp.float32),
            [pltpu.VMEM((BLK, EMB), jnp.float32)] * NBUF,
            [pltpu.VMEM((BLK // 2, 128), jnp.float32)] * NBUF,
            [pltpu.SemaphoreType.DMA] * NBUF,
            [pltpu.SemaphoreType.DMA] * NBUF,
        ],
    )
    out = run(table, idx, pe)
    return out.reshape(seq, batch, EMB)
